# baseline (speedup 1.0000x reference)
import functools

import jax
import jax.numpy as jnp
import numpy as np
from jax import lax
from jax.experimental import pallas as pl
from jax.experimental.pallas import tpu as pltpu
from jax.experimental.pallas import tpu_sc as plsc

_PRE = 6000
_NPROP = 1000
_THR = 0.7
_PAD = 6144
_ROWS = 48
_BIG = 1 << 30


def _nms_kernel(a_ref, d_ref, y1_ref, x1_ref, y2_ref, x2_ref, sc_ref, ak_ref):
    ay1, ax1, ay2, ax2 = (a_ref[0, c] for c in range(4))
    dy = d_ref[0, 0] * 0.1
    dx = d_ref[0, 1] * 0.1
    dh = d_ref[0, 2] * 0.2
    dw = d_ref[0, 3] * 0.2
    h = ay2 - ay1
    w = ax2 - ax1
    cy = ay1 + 0.5 * h + dy * h
    cx = ax1 + 0.5 * w + dx * w
    h = h * jnp.exp(dh)
    w = w * jnp.exp(dw)
    y1 = jnp.clip(cy - 0.5 * h, 0.0, 1.0)
    x1 = jnp.clip(cx - 0.5 * w, 0.0, 1.0)
    y2 = jnp.clip(cy - 0.5 * h + h, 0.0, 1.0)
    x2 = jnp.clip(cx - 0.5 * w + w, 0.0, 1.0)
    area = jnp.maximum(y2 - y1, 0.0) * jnp.maximum(x2 - x1, 0.0)

    for c, plane in enumerate((y1, x1, y2, x2, area)):
        sc_ref[:, 128 * c : 128 * c + 48] = jnp.transpose(plane)

    for ref in (y1_ref, x1_ref, y2_ref, x2_ref):
        ref[...] = jnp.zeros((1, 8, 128), jnp.float32)
    ak_ref[...] = jnp.zeros((8, 128), jnp.float32)

    row_o = lax.broadcasted_iota(jnp.int32, (8, 128), 0)
    col_o = lax.broadcasted_iota(jnp.int32, (8, 128), 1)
    idx_o = row_o * 128 + col_o
    lane = lax.broadcasted_iota(jnp.int32, (1, 640), 1)
    lane128 = lane & 127

    def cond(carry):
        p, t = carry
        return (p < _PRE) & (t < _NPROP)

    def body(carry):
        p, t = carry
        rv = sc_ref[pl.ds(p & 127, 1), :]
        masked = jnp.where(lane128 == p // 128, rv, 0.0)
        by1 = jnp.sum(masked[:, 0:128])
        bx1 = jnp.sum(masked[:, 128:256])
        by2 = jnp.sum(masked[:, 256:384])
        bx2 = jnp.sum(masked[:, 384:512])
        barea = jnp.sum(masked[:, 512:640])

        ky1 = y1_ref[0]
        kx1 = x1_ref[0]
        ky2 = y2_ref[0]
        kx2 = x2_ref[0]
        yy1 = jnp.maximum(by1, ky1)
        xx1 = jnp.maximum(bx1, kx1)
        yy2 = jnp.minimum(by2, ky2)
        xx2 = jnp.minimum(bx2, kx2)
        inter = jnp.maximum(yy2 - yy1, 0.0) * jnp.maximum(xx2 - xx1, 0.0)
        union = barea + ak_ref[...] - inter
        iou = inter / jnp.maximum(union, 1e-8)
        keep = jnp.max(iou) <= _THR

        accf = jnp.where(keep, 1.0, 0.0)
        onehot = (idx_o == t).astype(jnp.float32) * accf
        y1_ref[0] += onehot * by1
        x1_ref[0] += onehot * bx1
        y2_ref[0] += onehot * by2
        x2_ref[0] += onehot * bx2
        ak_ref[...] += onehot * barea
        return p + 1, t + jnp.where(keep, 1, 0)

    lax.while_loop(cond, body, (jnp.int32(0), jnp.int32(0)))


_SORTN = 32768
_SROWS = _SORTN // 128


def _sort_kernel(key_ref, idx_ref):
    key = key_ref[0]
    row = lax.broadcasted_iota(jnp.int32, (_SROWS, 128), 0)
    col = lax.broadcasted_iota(jnp.int32, (_SROWS, 128), 1)
    pos = row * 128 + col
    idx = pos
    k = 2
    while k <= _SORTN:
        j = k // 2
        while j >= 1:
            bitj = (pos & j) != 0
            hold_early = ((pos & k) == 0) == (~bitj)
            ax, s = (0, j // 128) if j >= 128 else (1, j)
            pk = jnp.where(bitj, jnp.roll(key, s, axis=ax), jnp.roll(key, -s, axis=ax))
            pi = jnp.where(bitj, jnp.roll(idx, s, axis=ax), jnp.roll(idx, -s, axis=ax))
            before = (pk > key) | ((pk == key) & (pi < idx))
            take = before == hold_early
            key = jnp.where(take, pk, key)
            idx = jnp.where(take, pi, idx)
            j //= 2
        k *= 2
    idx_ref[0] = idx


def _topk_indices(scores):
    pad = jnp.full((4, _SORTN - scores.shape[1]), -jnp.inf, jnp.float32)
    keys = jnp.concatenate([scores, pad], axis=1).reshape(4, _SROWS, 128)
    sorted_idx = pl.pallas_call(
        _sort_kernel,
        grid=(4,),
        in_specs=[pl.BlockSpec((1, _SROWS, 128), lambda b: (b, 0, 0))],
        out_specs=pl.BlockSpec((1, _SROWS, 128), lambda b: (b, 0, 0)),
        out_shape=jax.ShapeDtypeStruct((4, _SROWS, 128), jnp.int32),
    )(keys)
    return sorted_idx.reshape(4, _SORTN)[:, :_PAD]


_NCHUNK = _PAD // 128


def _sc_gather_body(flat_hbm, off_hbm, out_hbm, idx_v, rows_v, sem):
    wid = lax.axis_index("s") * 2 + lax.axis_index("c")
    pltpu.sync_copy(off_hbm.at[wid], idx_v)
    copies = [
        pltpu.make_async_copy(flat_hbm.at[idx_v.at[c]], rows_v.at[c], sem)
        for c in range(_NCHUNK)
    ]
    for cp in copies:
        cp.start()
    for cp in copies:
        cp.wait()
    pltpu.sync_copy(rows_v, out_hbm.at[wid])


@functools.cache
def _sc_gather():
    return pl.kernel(
        _sc_gather_body,
        out_type=jax.ShapeDtypeStruct((32, _NCHUNK, 128), jnp.float32),
        mesh=plsc.VectorSubcoreMesh(core_axis_name="c", subcore_axis_name="s"),
        scratch_types=[
            pltpu.VMEM((_NCHUNK, 128), jnp.int32),
            pltpu.VMEM((_NCHUNK, 128), jnp.float32),
            pltpu.SemaphoreType.DMA,
        ],
    )


def kernel(rpn_class, rpn_bbox, anchors):
    scores = rpn_class[:, :, 1]
    ix = _topk_indices(scores)

    planes = jnp.concatenate(
        [anchors.transpose(0, 2, 1), rpn_bbox.transpose(0, 2, 1)], axis=1
    )
    flat = planes.reshape(-1)
    base = (jnp.arange(4)[:, None] * 8 + jnp.arange(8)[None, :]) * 20000
    offs = (ix[:, None, :] + base[:, :, None]).astype(jnp.int32)
    offs = offs.reshape(32, _NCHUNK, 128)

    gathered = _sc_gather()(flat, offs)
    g = gathered.reshape(4, 8, _ROWS, 128)

    outs = pl.pallas_call(
        _nms_kernel,
        grid=(4,),
        in_specs=[
            pl.BlockSpec((1, 4, _ROWS, 128), lambda b: (b, 0, 0, 0)),
            pl.BlockSpec((1, 4, _ROWS, 128), lambda b: (b, 0, 0, 0)),
        ],
        out_specs=[pl.BlockSpec((1, 8, 128), lambda b: (b, 0, 0))] * 4,
        out_shape=[jax.ShapeDtypeStruct((4, 8, 128), jnp.float32)] * 4,
        scratch_shapes=[
            pltpu.VMEM((128, 640), jnp.float32),
            pltpu.VMEM((8, 128), jnp.float32),
        ],
    )(g[:, :4], g[:, 4:])
    planes = [o.reshape(4, 1024) for o in outs]
    return jnp.stack(planes, axis=-1)[:, :_NPROP, :]


# speedup vs baseline: 1.5025x; 1.5025x over previous
import functools

import jax
import jax.numpy as jnp
import numpy as np
from jax import lax
from jax.experimental import pallas as pl
from jax.experimental.pallas import tpu as pltpu
from jax.experimental.pallas import tpu_sc as plsc

_PRE = 6000
_NPROP = 1000
_THR = 0.7
_PAD = 6144
_ROWS = 48
_BIG = 1 << 30


def _nms_kernel(a_ref, d_ref, y1_ref, x1_ref, y2_ref, x2_ref, sc_ref, ak_ref):
    for b in range(4):
        ay1, ax1, ay2, ax2 = (a_ref[b, c] for c in range(4))
        dy = d_ref[b, 0] * 0.1
        dx = d_ref[b, 1] * 0.1
        dh = d_ref[b, 2] * 0.2
        dw = d_ref[b, 3] * 0.2
        h = ay2 - ay1
        w = ax2 - ax1
        cy = ay1 + 0.5 * h + dy * h
        cx = ax1 + 0.5 * w + dx * w
        h = h * jnp.exp(dh)
        w = w * jnp.exp(dw)
        y1 = jnp.clip(cy - 0.5 * h, 0.0, 1.0)
        x1 = jnp.clip(cx - 0.5 * w, 0.0, 1.0)
        y2 = jnp.clip(cy - 0.5 * h + h, 0.0, 1.0)
        x2 = jnp.clip(cx - 0.5 * w + w, 0.0, 1.0)
        area = jnp.maximum(y2 - y1, 0.0) * jnp.maximum(x2 - x1, 0.0)
        for c, plane in enumerate((y1, x1, y2, x2, area)):
            sc_ref[b, :, 128 * c : 128 * c + 48] = jnp.transpose(plane)

    for ref in (y1_ref, x1_ref, y2_ref, x2_ref):
        ref[...] = jnp.zeros((4, 8, 128), jnp.float32)
    ak_ref[...] = jnp.zeros((4, 8, 128), jnp.float32)

    row_o = lax.broadcasted_iota(jnp.int32, (8, 128), 0)
    col_o = lax.broadcasted_iota(jnp.int32, (8, 128), 1)
    idx_o = row_o * 128 + col_o
    lane = lax.broadcasted_iota(jnp.int32, (1, 640), 1)
    lane128 = lane & 127

    def cond(carry):
        ps, ts = carry
        alive = False
        for b in range(4):
            alive |= (ps[b] < _PRE) & (ts[b] < _NPROP)
        return alive

    def body(carry):
        ps, ts = carry
        new_ps, new_ts = [], []
        for b in range(4):
            p, t = ps[b], ts[b]
            active = (p < _PRE) & (t < _NPROP)
            rv = sc_ref[b, pl.ds(p & 127, 1), :]
            masked = jnp.where(lane128 == p // 128, rv, 0.0)
            by1 = jnp.sum(masked[:, 0:128])
            bx1 = jnp.sum(masked[:, 128:256])
            by2 = jnp.sum(masked[:, 256:384])
            bx2 = jnp.sum(masked[:, 384:512])
            barea = jnp.sum(masked[:, 512:640])

            yy1 = jnp.maximum(by1, y1_ref[b])
            xx1 = jnp.maximum(bx1, x1_ref[b])
            yy2 = jnp.minimum(by2, y2_ref[b])
            xx2 = jnp.minimum(bx2, x2_ref[b])
            inter = jnp.maximum(yy2 - yy1, 0.0) * jnp.maximum(xx2 - xx1, 0.0)
            union = barea + ak_ref[b] - inter
            iou = inter / jnp.maximum(union, 1e-8)
            keep = (jnp.max(iou) <= _THR) & active

            accf = jnp.where(keep, 1.0, 0.0)
            onehot = (idx_o == t).astype(jnp.float32) * accf
            y1_ref[b] += onehot * by1
            x1_ref[b] += onehot * bx1
            y2_ref[b] += onehot * by2
            x2_ref[b] += onehot * bx2
            ak_ref[b] += onehot * barea
            new_ps.append(jnp.where(active, p + 1, p))
            new_ts.append(t + jnp.where(keep, 1, 0))
        return tuple(new_ps), tuple(new_ts)

    z = jnp.int32(0)
    lax.while_loop(cond, body, ((z,) * 4, (z,) * 4))


_SORTN = 32768
_SROWS = _SORTN // 128


def _sort_kernel(key_ref, idx_ref):
    key = key_ref[0]
    row = lax.broadcasted_iota(jnp.int32, (_SROWS, 128), 0)
    col = lax.broadcasted_iota(jnp.int32, (_SROWS, 128), 1)
    pos = row * 128 + col
    idx = pos
    k = 2
    while k <= _SORTN:
        j = k // 2
        while j >= 1:
            bitj = (pos & j) != 0
            hold_early = ((pos & k) == 0) == (~bitj)
            ax, s = (0, j // 128) if j >= 128 else (1, j)
            pk = jnp.where(bitj, jnp.roll(key, s, axis=ax), jnp.roll(key, -s, axis=ax))
            pi = jnp.where(bitj, jnp.roll(idx, s, axis=ax), jnp.roll(idx, -s, axis=ax))
            before = (pk > key) | ((pk == key) & (pi < idx))
            take = before == hold_early
            key = jnp.where(take, pk, key)
            idx = jnp.where(take, pi, idx)
            j //= 2
        k *= 2
    idx_ref[0] = idx


def _topk_indices(scores):
    pad = jnp.full((4, _SORTN - scores.shape[1]), -jnp.inf, jnp.float32)
    keys = jnp.concatenate([scores, pad], axis=1).reshape(4, _SROWS, 128)
    sorted_idx = pl.pallas_call(
        _sort_kernel,
        grid=(4,),
        in_specs=[pl.BlockSpec((1, _SROWS, 128), lambda b: (b, 0, 0))],
        out_specs=pl.BlockSpec((1, _SROWS, 128), lambda b: (b, 0, 0)),
        out_shape=jax.ShapeDtypeStruct((4, _SROWS, 128), jnp.int32),
    )(keys)
    return sorted_idx.reshape(4, _SORTN)[:, :_PAD]


_NCHUNK = _PAD // 128


def _sc_gather_body(flat_hbm, off_hbm, out_hbm, idx_v, rows_v, sem):
    wid = lax.axis_index("s") * 2 + lax.axis_index("c")
    pltpu.sync_copy(off_hbm.at[wid], idx_v)
    copies = [
        pltpu.make_async_copy(flat_hbm.at[idx_v.at[c]], rows_v.at[c], sem)
        for c in range(_NCHUNK)
    ]
    for cp in copies:
        cp.start()
    for cp in copies:
        cp.wait()
    pltpu.sync_copy(rows_v, out_hbm.at[wid])


@functools.cache
def _sc_gather():
    return pl.kernel(
        _sc_gather_body,
        out_type=jax.ShapeDtypeStruct((32, _NCHUNK, 128), jnp.float32),
        mesh=plsc.VectorSubcoreMesh(core_axis_name="c", subcore_axis_name="s"),
        scratch_types=[
            pltpu.VMEM((_NCHUNK, 128), jnp.int32),
            pltpu.VMEM((_NCHUNK, 128), jnp.float32),
            pltpu.SemaphoreType.DMA,
        ],
    )


def kernel(rpn_class, rpn_bbox, anchors):
    scores = rpn_class[:, :, 1]
    ix = _topk_indices(scores)

    planes = jnp.concatenate(
        [anchors.transpose(0, 2, 1), rpn_bbox.transpose(0, 2, 1)], axis=1
    )
    flat = planes.reshape(-1)
    base = (jnp.arange(4)[:, None] * 8 + jnp.arange(8)[None, :]) * 20000
    offs = (ix[:, None, :] + base[:, :, None]).astype(jnp.int32)
    offs = offs.reshape(32, _NCHUNK, 128)

    gathered = _sc_gather()(flat, offs)
    g = gathered.reshape(4, 8, _ROWS, 128)

    outs = pl.pallas_call(
        _nms_kernel,
        out_shape=[jax.ShapeDtypeStruct((4, 8, 128), jnp.float32)] * 4,
        scratch_shapes=[
            pltpu.VMEM((4, 128, 640), jnp.float32),
            pltpu.VMEM((4, 8, 128), jnp.float32),
        ],
    )(g[:, :4], g[:, 4:])
    planes = [o.reshape(4, 1024) for o in outs]
    return jnp.stack(planes, axis=-1)[:, :_NPROP, :]
